# baseline (speedup 1.0000x reference)
import functools

import jax
import jax.numpy as jnp
from jax import lax
from jax.experimental import pallas as pl
from jax.experimental.pallas import tpu as pltpu
from jax.experimental.pallas import tpu_sc as plsc

N_NODES = 100000
N_EDGES = 3200000
FEAT = 20
HID = 32
OUT = 2

NC = 2
NS = 16
NW = NC * NS
NPAD = 100352
ROWS_PER_SUB = NPAD // NS
FH = 16
PK = NPAD * FH // 128
PK_PER_SUB = PK // NS

CHUNK = 512
E_PER_W = 100352
W_ITERS = E_PER_W // CHUNK
W_ITERS_LAST = (N_EDGES - (NW - 1) * E_PER_W) // CHUNK
E_PER_SUB = 200704
S_ITERS = E_PER_SUB // CHUNK
S_ITERS_LAST = (N_EDGES - (NS - 1) * E_PER_SUB) // CHUNK


_mesh = plsc.VectorSubcoreMesh(core_axis_name="c", subcore_axis_name="s")
_cparams = pltpu.CompilerParams(use_tc_tiling_on_sc=False,
                                needs_layout_passes=False)


@functools.partial(
    pl.kernel,
    out_type=jax.ShapeDtypeStruct((NC, NPAD, FH), jnp.float32),
    mesh=_mesh,
    compiler_params=_cparams,
    scratch_types=[
        pltpu.VMEM((1, CHUNK), jnp.int32),
        pltpu.VMEM((CHUNK,), jnp.float32),
        pltpu.VMEM((ROWS_PER_SUB,), jnp.float32),
        pltpu.VMEM((ROWS_PER_SUB, FH), jnp.float32),
        pltpu.VMEM_SHARED((NPAD,), jnp.float32),
    ],
)
def _sc_degree(dst2d, ones_hbm, zeros_hbm, out_hbm,
               dst_v, ones_v, slice_v, rep_v, acc_sh):
    cid = lax.axis_index("c")
    sid = lax.axis_index("s")
    wid = sid * NC + cid
    row0 = sid * ROWS_PER_SUB
    pltpu.sync_copy(zeros_hbm.at[pl.ds(row0, ROWS_PER_SUB)],
                    acc_sh.at[pl.ds(row0, ROWS_PER_SUB)])
    pltpu.sync_copy(ones_hbm, ones_v)
    plsc.subcore_barrier()

    nit = jnp.where(wid == NW - 1, W_ITERS_LAST, W_ITERS)

    @pl.loop(0, nit)
    def _(it):
        pltpu.sync_copy(dst2d.at[pl.ds(wid * W_ITERS + it, 1)], dst_v)
        pltpu.sync_copy(ones_v, acc_sh.at[dst_v.at[0]], add=True)

    plsc.subcore_barrier()
    pltpu.sync_copy(acc_sh.at[pl.ds(row0, ROWS_PER_SUB)], slice_v)
    iota16 = lax.iota(jnp.int32, 16)

    @pl.loop(0, ROWS_PER_SUB, step=16)
    def _(r0):
        vals = slice_v[pl.ds(r0, 16)]
        rows = iota16 + r0
        for j in range(FH):
            plsc.store_scatter(rep_v, [rows, jnp.full((16,), j, jnp.int32)],
                               vals)

    pltpu.sync_copy(rep_v,
                    out_hbm.at[cid].at[pl.ds(sid * ROWS_PER_SUB,
                                             ROWS_PER_SUB)])


IBLK = 8


def _edge_pipeline(src2d, dst2d, y2d, acc2d, row0, nit,
                   src8, dst8, rows_a, sem_a, rows_b, sem_b):
    nbl = nit // IBLK
    rem = nit - nbl * IBLK

    @pl.loop(0, nbl)
    def _(b):
        r = row0 + b * IBLK
        pltpu.sync_copy(src2d.at[pl.ds(r, IBLK)], src8)
        pltpu.sync_copy(dst2d.at[pl.ds(r, IBLK)], dst8)
        pltpu.async_copy(y2d.at[src8.at[0]], rows_a, sem_a)

        @pl.loop(0, IBLK // 2)
        def _(p):
            k0 = 2 * p
            k1 = k0 + 1
            pltpu.async_copy(y2d.at[src8.at[k1]], rows_b, sem_b)
            pltpu.make_async_copy(y2d.at[src8.at[k0]], rows_a, sem_a).wait()
            pltpu.sync_copy(rows_a, acc2d.at[dst8.at[k0]], add=True)

            @pl.when(k1 + 1 < IBLK)
            def _():
                pltpu.async_copy(y2d.at[src8.at[k1 + 1]], rows_a, sem_a)

            pltpu.make_async_copy(y2d.at[src8.at[k1]], rows_b, sem_b).wait()
            pltpu.sync_copy(rows_b, acc2d.at[dst8.at[k1]], add=True)

    @pl.loop(0, rem)
    def _(t):
        r = row0 + nbl * IBLK + t
        pltpu.sync_copy(src2d.at[pl.ds(r, 1)], src8.at[pl.ds(0, 1)])
        pltpu.sync_copy(dst2d.at[pl.ds(r, 1)], dst8.at[pl.ds(0, 1)])
        pltpu.async_copy(y2d.at[src8.at[0]], rows_a, sem_a).wait()
        pltpu.sync_copy(rows_a, acc2d.at[dst8.at[0]], add=True)


_AGG_SCRATCH = [
    pltpu.VMEM((IBLK, CHUNK), jnp.int32),
    pltpu.VMEM((IBLK, CHUNK), jnp.int32),
    pltpu.VMEM((CHUNK, FH), jnp.float32),
    pltpu.SemaphoreType.DMA,
    pltpu.VMEM((CHUNK, FH), jnp.float32),
    pltpu.SemaphoreType.DMA,
    pltpu.VMEM_SHARED((NPAD, FH), jnp.float32),
]


def _agg_epilogue(acc_sh, out_hbm, cid, sid):
    plsc.subcore_barrier()
    pltpu.sync_copy(
        acc_sh.at[pl.ds(sid * ROWS_PER_SUB, ROWS_PER_SUB)],
        out_hbm.at[cid].at[pl.ds(sid * ROWS_PER_SUB, ROWS_PER_SUB)])


def _agg_init(zeros_hbm, acc_sh, sid):
    pltpu.sync_copy(
        zeros_hbm.at[pl.ds(sid * ROWS_PER_SUB, ROWS_PER_SUB)],
        acc_sh.at[pl.ds(sid * ROWS_PER_SUB, ROWS_PER_SUB)])
    plsc.subcore_barrier()


@functools.partial(
    pl.kernel,
    out_type=jax.ShapeDtypeStruct((NC, NPAD, FH), jnp.float32),
    mesh=_mesh,
    compiler_params=_cparams,
    scratch_types=_AGG_SCRATCH,
)
def _sc_agg_l1(src2d, dst2d, ya_hbm, yb_hbm, zeros_hbm, out_hbm,
               src8, dst8, rows_a, sem_a, rows_b, sem_b, acc_sh):
    cid = lax.axis_index("c")
    sid = lax.axis_index("s")
    _agg_init(zeros_hbm, acc_sh, sid)

    nit = jnp.where(sid == NS - 1, S_ITERS_LAST, S_ITERS)
    bufs = (src8, dst8, rows_a, sem_a, rows_b, sem_b)

    @pl.when(cid == 0)
    def _():
        _edge_pipeline(src2d, dst2d, ya_hbm, acc_sh, sid * S_ITERS, nit,
                       *bufs)

    @pl.when(cid == 1)
    def _():
        _edge_pipeline(src2d, dst2d, yb_hbm, acc_sh, sid * S_ITERS, nit,
                       *bufs)

    _agg_epilogue(acc_sh, out_hbm, cid, sid)


@functools.partial(
    pl.kernel,
    out_type=jax.ShapeDtypeStruct((NC, NPAD, FH), jnp.float32),
    mesh=_mesh,
    compiler_params=_cparams,
    scratch_types=_AGG_SCRATCH,
)
def _sc_agg_l2(src2d, dst2d, y_hbm, zeros_hbm, out_hbm,
               src8, dst8, rows_a, sem_a, rows_b, sem_b, acc_sh):
    cid = lax.axis_index("c")
    sid = lax.axis_index("s")
    wid = sid * NC + cid
    _agg_init(zeros_hbm, acc_sh, sid)

    nit = jnp.where(wid == NW - 1, W_ITERS_LAST, W_ITERS)
    _edge_pipeline(src2d, dst2d, y_hbm, acc_sh, wid * W_ITERS, nit,
                   src8, dst8, rows_a, sem_a, rows_b, sem_b)

    _agg_epilogue(acc_sh, out_hbm, cid, sid)


BLK = 7168
BLKP = BLK // 8
TGRID = NPAD // BLK


def _nrow_spec(w):
    return pl.BlockSpec((BLK, w), lambda i: (i, 0))


def _prow_spec():
    return pl.BlockSpec((BLKP, 128), lambda i: (i, 0))


def _pair_spec():
    return pl.BlockSpec((NC, BLKP, 128), lambda i: (0, i, 0))


def _rep_spec(shape):
    return pl.BlockSpec(shape, lambda i: tuple(0 for _ in shape))


def _tc_prep_body(deg_ref, xpa_ref, xpb_ref, dinv_ref, xa_ref, xb_ref):
    dinv = lax.rsqrt(deg_ref[0] + deg_ref[1] + 1.0)
    dinv_ref[...] = dinv
    xa_ref[...] = xpa_ref[...] * dinv
    xb_ref[...] = xpb_ref[...] * dinv


_tc_prep = pl.pallas_call(
    _tc_prep_body,
    grid=(TGRID,),
    in_specs=[_pair_spec(), _prow_spec(), _prow_spec()],
    out_specs=[_prow_spec(), _prow_spec(), _prow_spec()],
    out_shape=[jax.ShapeDtypeStruct((PK, 128), jnp.float32)] * 3,
)


def _tc_dense_body(acc_ref, xa_ref, xb_ref, dinv_ref, w1_ref, b1_ref,
                   w2_ref, o_ref):
    dinv = dinv_ref[...]
    za = (acc_ref[0] + xa_ref[...]) * dinv
    zb = (acc_ref[1] + xb_ref[...]) * dinv
    z = jnp.concatenate([za, zb], axis=1)
    h1 = jnp.maximum(
        jnp.dot(z, w1_ref[...], preferred_element_type=jnp.float32)
        + b1_ref[...], 0.0)
    h2 = jnp.dot(h1, w2_ref[...], preferred_element_type=jnp.float32)
    o_ref[...] = h2 * dinv


_tc_dense = pl.pallas_call(
    _tc_dense_body,
    grid=(TGRID,),
    in_specs=[_pair_spec(), _prow_spec(), _prow_spec(), _prow_spec(),
              _rep_spec((256, 256)), _rep_spec((1, 256)),
              _rep_spec((256, 128))],
    out_specs=_prow_spec(),
    out_shape=jax.ShapeDtypeStruct((PK, 128), jnp.float32),
)


def _tc_final_body(acc_ref, h2s_ref, dinv_ref, b2_ref, o_ref):
    o_ref[...] = ((acc_ref[0] + acc_ref[1] + h2s_ref[...])
                  * dinv_ref[...] + b2_ref[...])


_tc_final = pl.pallas_call(
    _tc_final_body,
    grid=(TGRID,),
    in_specs=[_pair_spec(), _prow_spec(), _prow_spec(),
              _rep_spec((1, 128))],
    out_specs=_prow_spec(),
    out_shape=jax.ShapeDtypeStruct((PK, 128), jnp.float32),
)


def kernel(x, edge_index, W1, b1, W2, b2):
    src = edge_index[0].astype(jnp.int32)
    dst = edge_index[1].astype(jnp.int32)
    src2d = src.reshape(N_EDGES // CHUNK, CHUNK)
    dst2d = dst.reshape(N_EDGES // CHUNK, CHUNK)

    ones_chunk = jnp.ones((CHUNK,), jnp.float32)
    z1 = jnp.zeros((NPAD,), jnp.float32)
    zn = jnp.zeros((NPAD, FH), jnp.float32)

    NPK = N_NODES * FH // 128
    xpa = jnp.pad(x[:, :FH].reshape(NPK, 128), ((0, PK - NPK), (0, 0)))
    xpb = jnp.pad(x[:, FEAT - FH:].reshape(NPK, 128),
                  ((0, PK - NPK), (0, 0)))

    deg16 = _sc_degree(dst2d, ones_chunk, z1)
    dinv16, xa, xb = _tc_prep(deg16.reshape(NC, PK, 128), xpa, xpb)

    acc1 = _sc_agg_l1(src2d, dst2d, xa.reshape(NPAD, FH),
                      xb.reshape(NPAD, FH), zn)

    w1p = jnp.concatenate(
        [W1[:FH], jnp.zeros((2 * FH - FEAT, HID), W1.dtype), W1[FH:]],
        axis=0)
    w2p = jnp.pad(W2, ((0, 0), (0, FH - OUT)))
    eye8 = jnp.eye(8, dtype=jnp.float32)
    w1big = jnp.einsum("hfgp,kK->hkfgKp", w1p.reshape(2, FH, 2, FH),
                       eye8).reshape(256, 256)
    w2big = jnp.einsum("gpo,kK->gkpKo", w2p.reshape(2, FH, FH),
                       eye8).reshape(256, 128)
    b1big = jnp.broadcast_to(b1.reshape(2, 1, FH), (2, 8, FH)).reshape(1, 256)
    b2big = jnp.broadcast_to(jnp.pad(b2, (0, FH - OUT)).reshape(1, 1, FH),
                             (1, 8, FH)).reshape(1, 128)

    h2s = _tc_dense(acc1.reshape(NC, PK, 128), xa, xb, dinv16,
                    w1big, b1big, w2big)

    acc2 = _sc_agg_l2(src2d, dst2d, h2s.reshape(NPAD, FH), zn)

    outp = _tc_final(acc2.reshape(NC, PK, 128), h2s, dinv16, b2big)
    return (outp[:NPK].reshape(NPK, 8, FH)[:, :, :OUT]
            .reshape(N_NODES, OUT))


# speedup vs baseline: 1.0494x; 1.0494x over previous
import functools

import jax
import jax.numpy as jnp
from jax import lax
from jax.experimental import pallas as pl
from jax.experimental.pallas import tpu as pltpu
from jax.experimental.pallas import tpu_sc as plsc

N_NODES = 100000
N_EDGES = 3200000
FEAT = 20
HID = 32
OUT = 2

NC = 2
NS = 16
NW = NC * NS
NPAD = 100352
ROWS_PER_SUB = NPAD // NS
FH = 16
PK = NPAD * FH // 128
PK_PER_SUB = PK // NS

CHUNK = 512
E_PER_W = 100352
W_ITERS = E_PER_W // CHUNK
W_ITERS_LAST = (N_EDGES - (NW - 1) * E_PER_W) // CHUNK
E_PER_SUB = 200704
S_ITERS = E_PER_SUB // CHUNK
S_ITERS_LAST = (N_EDGES - (NS - 1) * E_PER_SUB) // CHUNK


_mesh = plsc.VectorSubcoreMesh(core_axis_name="c", subcore_axis_name="s")
_cparams = pltpu.CompilerParams(use_tc_tiling_on_sc=False,
                                needs_layout_passes=False)


@functools.partial(
    pl.kernel,
    out_type=jax.ShapeDtypeStruct((NC, NPAD, FH), jnp.float32),
    mesh=_mesh,
    compiler_params=_cparams,
    scratch_types=[
        pltpu.VMEM((8, CHUNK), jnp.int32),
        pltpu.VMEM((CHUNK,), jnp.float32),
        pltpu.VMEM((ROWS_PER_SUB,), jnp.float32),
        pltpu.VMEM((ROWS_PER_SUB, FH), jnp.float32),
        pltpu.VMEM_SHARED((NPAD,), jnp.float32),
    ],
)
def _sc_degree(dst2d, ones_hbm, zeros_hbm, out_hbm,
               dst_v, ones_v, slice_v, rep_v, acc_sh):
    cid = lax.axis_index("c")
    sid = lax.axis_index("s")
    wid = sid * NC + cid
    row0 = sid * ROWS_PER_SUB
    pltpu.sync_copy(zeros_hbm.at[pl.ds(row0, ROWS_PER_SUB)],
                    acc_sh.at[pl.ds(row0, ROWS_PER_SUB)])
    pltpu.sync_copy(ones_hbm, ones_v)
    plsc.subcore_barrier()

    nit = jnp.where(wid == NW - 1, W_ITERS_LAST, W_ITERS)
    nbl = nit // 8

    @pl.loop(0, nbl)
    def _(b):
        pltpu.sync_copy(dst2d.at[pl.ds(wid * W_ITERS + b * 8, 8)], dst_v)

        @pl.loop(0, 8)
        def _(k):
            pltpu.sync_copy(ones_v, acc_sh.at[dst_v.at[k]], add=True)

    @pl.loop(0, nit - nbl * 8)
    def _(t):
        pltpu.sync_copy(dst2d.at[pl.ds(wid * W_ITERS + nbl * 8 + t, 1)],
                        dst_v.at[pl.ds(0, 1)])
        pltpu.sync_copy(ones_v, acc_sh.at[dst_v.at[0]], add=True)

    plsc.subcore_barrier()
    pltpu.sync_copy(acc_sh.at[pl.ds(row0, ROWS_PER_SUB)], slice_v)
    iota16 = lax.iota(jnp.int32, 16)

    @pl.loop(0, ROWS_PER_SUB, step=16)
    def _(r0):
        vals = slice_v[pl.ds(r0, 16)]
        rows = iota16 + r0
        for j in range(FH):
            plsc.store_scatter(rep_v, [rows, jnp.full((16,), j, jnp.int32)],
                               vals)

    pltpu.sync_copy(rep_v,
                    out_hbm.at[cid].at[pl.ds(sid * ROWS_PER_SUB,
                                             ROWS_PER_SUB)])


IBLK = 8


def _edge_pipeline(src2d, dst2d, y2d, acc2d, row0, nit,
                   src8, dst8, rows_a, sem_a, rows_b, sem_b):
    nbl = nit // IBLK
    rem = nit - nbl * IBLK

    @pl.loop(0, nbl)
    def _(b):
        r = row0 + b * IBLK
        pltpu.sync_copy(src2d.at[pl.ds(r, IBLK)], src8)
        pltpu.sync_copy(dst2d.at[pl.ds(r, IBLK)], dst8)
        pltpu.async_copy(y2d.at[src8.at[0]], rows_a, sem_a)

        @pl.loop(0, IBLK // 2)
        def _(p):
            k0 = 2 * p
            k1 = k0 + 1
            pltpu.async_copy(y2d.at[src8.at[k1]], rows_b, sem_b)
            pltpu.make_async_copy(y2d.at[src8.at[k0]], rows_a, sem_a).wait()
            pltpu.sync_copy(rows_a, acc2d.at[dst8.at[k0]], add=True)

            @pl.when(k1 + 1 < IBLK)
            def _():
                pltpu.async_copy(y2d.at[src8.at[k1 + 1]], rows_a, sem_a)

            pltpu.make_async_copy(y2d.at[src8.at[k1]], rows_b, sem_b).wait()
            pltpu.sync_copy(rows_b, acc2d.at[dst8.at[k1]], add=True)

    @pl.loop(0, rem)
    def _(t):
        r = row0 + nbl * IBLK + t
        pltpu.sync_copy(src2d.at[pl.ds(r, 1)], src8.at[pl.ds(0, 1)])
        pltpu.sync_copy(dst2d.at[pl.ds(r, 1)], dst8.at[pl.ds(0, 1)])
        pltpu.async_copy(y2d.at[src8.at[0]], rows_a, sem_a).wait()
        pltpu.sync_copy(rows_a, acc2d.at[dst8.at[0]], add=True)


_AGG_SCRATCH = [
    pltpu.VMEM((IBLK, CHUNK), jnp.int32),
    pltpu.VMEM((IBLK, CHUNK), jnp.int32),
    pltpu.VMEM((CHUNK, FH), jnp.float32),
    pltpu.SemaphoreType.DMA,
    pltpu.VMEM((CHUNK, FH), jnp.float32),
    pltpu.SemaphoreType.DMA,
    pltpu.VMEM_SHARED((NPAD, FH), jnp.float32),
]


def _agg_epilogue(acc_sh, out_hbm, cid, sid):
    plsc.subcore_barrier()
    pltpu.sync_copy(
        acc_sh.at[pl.ds(sid * ROWS_PER_SUB, ROWS_PER_SUB)],
        out_hbm.at[cid].at[pl.ds(sid * ROWS_PER_SUB, ROWS_PER_SUB)])


def _agg_init(zeros_hbm, acc_sh, sid):
    pltpu.sync_copy(
        zeros_hbm.at[pl.ds(sid * ROWS_PER_SUB, ROWS_PER_SUB)],
        acc_sh.at[pl.ds(sid * ROWS_PER_SUB, ROWS_PER_SUB)])
    plsc.subcore_barrier()


@functools.partial(
    pl.kernel,
    out_type=jax.ShapeDtypeStruct((NC, NPAD, FH), jnp.float32),
    mesh=_mesh,
    compiler_params=_cparams,
    scratch_types=_AGG_SCRATCH,
)
def _sc_agg_l1(src2d, dst2d, ya_hbm, yb_hbm, zeros_hbm, out_hbm,
               src8, dst8, rows_a, sem_a, rows_b, sem_b, acc_sh):
    cid = lax.axis_index("c")
    sid = lax.axis_index("s")
    _agg_init(zeros_hbm, acc_sh, sid)

    nit = jnp.where(sid == NS - 1, S_ITERS_LAST, S_ITERS)
    bufs = (src8, dst8, rows_a, sem_a, rows_b, sem_b)

    @pl.when(cid == 0)
    def _():
        _edge_pipeline(src2d, dst2d, ya_hbm, acc_sh, sid * S_ITERS, nit,
                       *bufs)

    @pl.when(cid == 1)
    def _():
        _edge_pipeline(src2d, dst2d, yb_hbm, acc_sh, sid * S_ITERS, nit,
                       *bufs)

    _agg_epilogue(acc_sh, out_hbm, cid, sid)


@functools.partial(
    pl.kernel,
    out_type=jax.ShapeDtypeStruct((NC, NPAD, FH), jnp.float32),
    mesh=_mesh,
    compiler_params=_cparams,
    scratch_types=_AGG_SCRATCH,
)
def _sc_agg_l2(src2d, dst2d, y_hbm, zeros_hbm, out_hbm,
               src8, dst8, rows_a, sem_a, rows_b, sem_b, acc_sh):
    cid = lax.axis_index("c")
    sid = lax.axis_index("s")
    wid = sid * NC + cid
    _agg_init(zeros_hbm, acc_sh, sid)

    nit = jnp.where(wid == NW - 1, W_ITERS_LAST, W_ITERS)
    _edge_pipeline(src2d, dst2d, y_hbm, acc_sh, wid * W_ITERS, nit,
                   src8, dst8, rows_a, sem_a, rows_b, sem_b)

    _agg_epilogue(acc_sh, out_hbm, cid, sid)


BLK = 7168
BLKP = BLK // 8
TGRID = NPAD // BLK


def _nrow_spec(w):
    return pl.BlockSpec((BLK, w), lambda i: (i, 0))


def _prow_spec():
    return pl.BlockSpec((BLKP, 128), lambda i: (i, 0))


def _pair_spec():
    return pl.BlockSpec((NC, BLKP, 128), lambda i: (0, i, 0))


def _rep_spec(shape):
    return pl.BlockSpec(shape, lambda i: tuple(0 for _ in shape))


def _tc_prep_body(deg_ref, xpa_ref, xpb_ref, dinv_ref, xa_ref, xb_ref):
    dinv = lax.rsqrt(deg_ref[0] + deg_ref[1] + 1.0)
    dinv_ref[...] = dinv
    xa_ref[...] = xpa_ref[...] * dinv
    xb_ref[...] = xpb_ref[...] * dinv


_tc_prep = pl.pallas_call(
    _tc_prep_body,
    grid=(TGRID,),
    in_specs=[_pair_spec(), _prow_spec(), _prow_spec()],
    out_specs=[_prow_spec(), _prow_spec(), _prow_spec()],
    out_shape=[jax.ShapeDtypeStruct((PK, 128), jnp.float32)] * 3,
)


def _tc_dense_body(acc_ref, xa_ref, xb_ref, dinv_ref, w1_ref, b1_ref,
                   w2_ref, o_ref):
    dinv = dinv_ref[...]
    za = (acc_ref[0] + xa_ref[...]) * dinv
    zb = (acc_ref[1] + xb_ref[...]) * dinv
    z = jnp.concatenate([za, zb], axis=1)
    h1 = jnp.maximum(
        jnp.dot(z, w1_ref[...], preferred_element_type=jnp.float32)
        + b1_ref[...], 0.0)
    h2 = jnp.dot(h1, w2_ref[...], preferred_element_type=jnp.float32)
    o_ref[...] = h2 * dinv


_tc_dense = pl.pallas_call(
    _tc_dense_body,
    grid=(TGRID,),
    in_specs=[_pair_spec(), _prow_spec(), _prow_spec(), _prow_spec(),
              _rep_spec((256, 256)), _rep_spec((1, 256)),
              _rep_spec((256, 128))],
    out_specs=_prow_spec(),
    out_shape=jax.ShapeDtypeStruct((PK, 128), jnp.float32),
)


def _tc_final_body(acc_ref, h2s_ref, dinv_ref, b2_ref, o_ref):
    o_ref[...] = ((acc_ref[0] + acc_ref[1] + h2s_ref[...])
                  * dinv_ref[...] + b2_ref[...])


_tc_final = pl.pallas_call(
    _tc_final_body,
    grid=(TGRID,),
    in_specs=[_pair_spec(), _prow_spec(), _prow_spec(),
              _rep_spec((1, 128))],
    out_specs=_prow_spec(),
    out_shape=jax.ShapeDtypeStruct((PK, 128), jnp.float32),
)


def kernel(x, edge_index, W1, b1, W2, b2):
    src = edge_index[0].astype(jnp.int32)
    dst = edge_index[1].astype(jnp.int32)
    src2d = src.reshape(N_EDGES // CHUNK, CHUNK)
    dst2d = dst.reshape(N_EDGES // CHUNK, CHUNK)

    ones_chunk = jnp.ones((CHUNK,), jnp.float32)
    z1 = jnp.zeros((NPAD,), jnp.float32)
    zn = jnp.zeros((NPAD, FH), jnp.float32)

    NPK = N_NODES * FH // 128
    xpa = jnp.pad(x[:, :FH].reshape(NPK, 128), ((0, PK - NPK), (0, 0)))
    xpb = jnp.pad(x[:, FEAT - FH:].reshape(NPK, 128),
                  ((0, PK - NPK), (0, 0)))

    deg16 = _sc_degree(dst2d, ones_chunk, z1)
    dinv16, xa, xb = _tc_prep(deg16.reshape(NC, PK, 128), xpa, xpb)

    acc1 = _sc_agg_l1(src2d, dst2d, xa.reshape(NPAD, FH),
                      xb.reshape(NPAD, FH), zn)

    w1p = jnp.concatenate(
        [W1[:FH], jnp.zeros((2 * FH - FEAT, HID), W1.dtype), W1[FH:]],
        axis=0)
    w2p = jnp.pad(W2, ((0, 0), (0, FH - OUT)))
    eye8 = jnp.eye(8, dtype=jnp.float32)
    w1big = jnp.einsum("hfgp,kK->hkfgKp", w1p.reshape(2, FH, 2, FH),
                       eye8).reshape(256, 256)
    w2big = jnp.einsum("gpo,kK->gkpKo", w2p.reshape(2, FH, FH),
                       eye8).reshape(256, 128)
    b1big = jnp.broadcast_to(b1.reshape(2, 1, FH), (2, 8, FH)).reshape(1, 256)
    b2big = jnp.broadcast_to(jnp.pad(b2, (0, FH - OUT)).reshape(1, 1, FH),
                             (1, 8, FH)).reshape(1, 128)

    h2s = _tc_dense(acc1.reshape(NC, PK, 128), xa, xb, dinv16,
                    w1big, b1big, w2big)

    acc2 = _sc_agg_l2(src2d, dst2d, h2s.reshape(NPAD, FH), zn)

    outp = _tc_final(acc2.reshape(NC, PK, 128), h2s, dinv16, b2big)
    return (outp[:NPK].reshape(NPK, 8, FH)[:, :, :OUT]
            .reshape(N_NODES, OUT))


# speedup vs baseline: 1.0506x; 1.0011x over previous
import functools

import jax
import jax.numpy as jnp
from jax import lax
from jax.experimental import pallas as pl
from jax.experimental.pallas import tpu as pltpu
from jax.experimental.pallas import tpu_sc as plsc

N_NODES = 100000
N_EDGES = 3200000
FEAT = 20
HID = 32
OUT = 2

NC = 2
NS = 16
NW = NC * NS
NPAD = 100352
ROWS_PER_SUB = NPAD // NS
FH = 16
PK = NPAD * FH // 128

CHUNK = 512
E_PER_W = 100352
W_ITERS = E_PER_W // CHUNK
W_ITERS_LAST = (N_EDGES - (NW - 1) * E_PER_W) // CHUNK
E_PER_SUB = 200704
S_ITERS = E_PER_SUB // CHUNK
S_ITERS_LAST = (N_EDGES - (NS - 1) * E_PER_SUB) // CHUNK


_mesh = plsc.VectorSubcoreMesh(core_axis_name="c", subcore_axis_name="s")
_cparams = pltpu.CompilerParams(use_tc_tiling_on_sc=False,
                                needs_layout_passes=False)


@functools.partial(
    pl.kernel,
    out_type=jax.ShapeDtypeStruct((NC, NPAD, FH), jnp.float32),
    mesh=_mesh,
    compiler_params=_cparams,
    scratch_types=[
        pltpu.VMEM((8, CHUNK), jnp.int32),
        pltpu.VMEM((CHUNK,), jnp.float32),
        pltpu.VMEM((ROWS_PER_SUB,), jnp.float32),
        pltpu.VMEM((ROWS_PER_SUB, FH), jnp.float32),
        pltpu.VMEM_SHARED((NPAD,), jnp.float32),
    ],
)
def _sc_degree(dst2d, ones_hbm, zeros_hbm, out_hbm,
               dst_v, ones_v, slice_v, rep_v, acc_sh):
    cid = lax.axis_index("c")
    sid = lax.axis_index("s")
    wid = sid * NC + cid
    row0 = sid * ROWS_PER_SUB
    pltpu.sync_copy(zeros_hbm.at[pl.ds(row0, ROWS_PER_SUB)],
                    acc_sh.at[pl.ds(row0, ROWS_PER_SUB)])
    pltpu.sync_copy(ones_hbm, ones_v)
    plsc.subcore_barrier()

    nit = jnp.where(wid == NW - 1, W_ITERS_LAST, W_ITERS)
    nbl = nit // 8

    @pl.loop(0, nbl)
    def _(b):
        pltpu.sync_copy(dst2d.at[pl.ds(wid * W_ITERS + b * 8, 8)], dst_v)

        @pl.loop(0, 8)
        def _(k):
            pltpu.sync_copy(ones_v, acc_sh.at[dst_v.at[k]], add=True)

    @pl.loop(0, nit - nbl * 8)
    def _(t):
        pltpu.sync_copy(dst2d.at[pl.ds(wid * W_ITERS + nbl * 8 + t, 1)],
                        dst_v.at[pl.ds(0, 1)])
        pltpu.sync_copy(ones_v, acc_sh.at[dst_v.at[0]], add=True)

    plsc.subcore_barrier()
    pltpu.sync_copy(acc_sh.at[pl.ds(row0, ROWS_PER_SUB)], slice_v)
    iota16 = lax.iota(jnp.int32, 16)

    @pl.loop(0, ROWS_PER_SUB, step=16)
    def _(r0):
        vals = slice_v[pl.ds(r0, 16)]
        rows = iota16 + r0
        for j in range(FH):
            plsc.store_scatter(rep_v, [rows, jnp.full((16,), j, jnp.int32)],
                               vals)

    pltpu.sync_copy(rep_v,
                    out_hbm.at[cid].at[pl.ds(sid * ROWS_PER_SUB,
                                             ROWS_PER_SUB)])


IBLK = 8


def _edge_pipeline(src2d, dst2d, y2d, acc2d, row0, nit,
                   src8, dst8, rows_a, sem_a, rows_b, sem_b):
    nbl = nit // IBLK
    rem = nit - nbl * IBLK

    @pl.loop(0, nbl)
    def _(b):
        r = row0 + b * IBLK
        pltpu.sync_copy(src2d.at[pl.ds(r, IBLK)], src8)
        pltpu.sync_copy(dst2d.at[pl.ds(r, IBLK)], dst8)
        pltpu.async_copy(y2d.at[src8.at[0]], rows_a, sem_a)

        @pl.loop(0, IBLK // 2)
        def _(p):
            k0 = 2 * p
            k1 = k0 + 1
            pltpu.async_copy(y2d.at[src8.at[k1]], rows_b, sem_b)
            pltpu.make_async_copy(y2d.at[src8.at[k0]], rows_a, sem_a).wait()
            pltpu.sync_copy(rows_a, acc2d.at[dst8.at[k0]], add=True)

            @pl.when(k1 + 1 < IBLK)
            def _():
                pltpu.async_copy(y2d.at[src8.at[k1 + 1]], rows_a, sem_a)

            pltpu.make_async_copy(y2d.at[src8.at[k1]], rows_b, sem_b).wait()
            pltpu.sync_copy(rows_b, acc2d.at[dst8.at[k1]], add=True)

    @pl.loop(0, rem)
    def _(t):
        r = row0 + nbl * IBLK + t
        pltpu.sync_copy(src2d.at[pl.ds(r, 1)], src8.at[pl.ds(0, 1)])
        pltpu.sync_copy(dst2d.at[pl.ds(r, 1)], dst8.at[pl.ds(0, 1)])
        pltpu.async_copy(y2d.at[src8.at[0]], rows_a, sem_a).wait()
        pltpu.sync_copy(rows_a, acc2d.at[dst8.at[0]], add=True)


_AGG_SCRATCH = [
    pltpu.VMEM((IBLK, CHUNK), jnp.int32),
    pltpu.VMEM((IBLK, CHUNK), jnp.int32),
    pltpu.VMEM((CHUNK, FH), jnp.float32),
    pltpu.SemaphoreType.DMA,
    pltpu.VMEM((CHUNK, FH), jnp.float32),
    pltpu.SemaphoreType.DMA,
    pltpu.VMEM_SHARED((NPAD, FH), jnp.float32),
]


def _agg_epilogue(acc_sh, out_hbm, cid, sid):
    plsc.subcore_barrier()
    pltpu.sync_copy(
        acc_sh.at[pl.ds(sid * ROWS_PER_SUB, ROWS_PER_SUB)],
        out_hbm.at[cid].at[pl.ds(sid * ROWS_PER_SUB, ROWS_PER_SUB)])


def _agg_init(zeros_hbm, acc_sh, sid):
    pltpu.sync_copy(
        zeros_hbm.at[pl.ds(sid * ROWS_PER_SUB, ROWS_PER_SUB)],
        acc_sh.at[pl.ds(sid * ROWS_PER_SUB, ROWS_PER_SUB)])
    plsc.subcore_barrier()


@functools.partial(
    pl.kernel,
    out_type=jax.ShapeDtypeStruct((NC, NPAD, FH), jnp.float32),
    mesh=_mesh,
    compiler_params=_cparams,
    scratch_types=_AGG_SCRATCH,
)
def _sc_agg_l1(src2d, dst2d, ya_hbm, yb_hbm, zeros_hbm, out_hbm,
               src8, dst8, rows_a, sem_a, rows_b, sem_b, acc_sh):
    cid = lax.axis_index("c")
    sid = lax.axis_index("s")
    _agg_init(zeros_hbm, acc_sh, sid)

    nit = jnp.where(sid == NS - 1, S_ITERS_LAST, S_ITERS)
    bufs = (src8, dst8, rows_a, sem_a, rows_b, sem_b)

    @pl.when(cid == 0)
    def _():
        _edge_pipeline(src2d, dst2d, ya_hbm, acc_sh, sid * S_ITERS, nit,
                       *bufs)

    @pl.when(cid == 1)
    def _():
        _edge_pipeline(src2d, dst2d, yb_hbm, acc_sh, sid * S_ITERS, nit,
                       *bufs)

    _agg_epilogue(acc_sh, out_hbm, cid, sid)


@functools.partial(
    pl.kernel,
    out_type=jax.ShapeDtypeStruct((NC, NPAD, FH), jnp.float32),
    mesh=_mesh,
    compiler_params=_cparams,
    scratch_types=_AGG_SCRATCH,
)
def _sc_agg_l2(src2d, dst2d, y_hbm, zeros_hbm, out_hbm,
               src8, dst8, rows_a, sem_a, rows_b, sem_b, acc_sh):
    cid = lax.axis_index("c")
    sid = lax.axis_index("s")
    wid = sid * NC + cid
    _agg_init(zeros_hbm, acc_sh, sid)

    nit = jnp.where(wid == NW - 1, W_ITERS_LAST, W_ITERS)
    _edge_pipeline(src2d, dst2d, y_hbm, acc_sh, wid * W_ITERS, nit,
                   src8, dst8, rows_a, sem_a, rows_b, sem_b)

    _agg_epilogue(acc_sh, out_hbm, cid, sid)


BLK = 7168
BLKP = BLK // 8
TGRID = NPAD // BLK


def _prow_spec():
    return pl.BlockSpec((BLKP, 128), lambda i: (i, 0))


def _pair_spec():
    return pl.BlockSpec((NC, BLKP, 128), lambda i: (0, i, 0))


def _rep_spec(shape):
    return pl.BlockSpec(shape, lambda i: tuple(0 for _ in shape))


def _tc_prep_body(deg_ref, xpa_ref, xpb_ref, dinv_ref, xa_ref, xb_ref):
    dinv = lax.rsqrt(deg_ref[0] + deg_ref[1] + 1.0)
    dinv_ref[...] = dinv
    xa_ref[...] = xpa_ref[...] * dinv
    xb_ref[...] = xpb_ref[...] * dinv


_tc_prep = pl.pallas_call(
    _tc_prep_body,
    grid=(TGRID,),
    in_specs=[_pair_spec(), _prow_spec(), _prow_spec()],
    out_specs=[_prow_spec(), _prow_spec(), _prow_spec()],
    out_shape=[jax.ShapeDtypeStruct((PK, 128), jnp.float32)] * 3,
)


def _tc_dense_body(acc_ref, xa_ref, xb_ref, dinv_ref, w1_ref, b1_ref,
                   w2_ref, o_ref):
    dinv = dinv_ref[...]
    za = (acc_ref[0] + xa_ref[...]) * dinv
    zb = (acc_ref[1] + xb_ref[...]) * dinv
    z = jnp.concatenate([za, zb], axis=1)
    h1 = jnp.maximum(
        jnp.dot(z, w1_ref[...], preferred_element_type=jnp.float32)
        + b1_ref[...], 0.0)
    h2 = jnp.dot(h1, w2_ref[...], preferred_element_type=jnp.float32)
    o_ref[...] = h2 * dinv


_tc_dense = pl.pallas_call(
    _tc_dense_body,
    grid=(TGRID,),
    in_specs=[_pair_spec(), _prow_spec(), _prow_spec(), _prow_spec(),
              _rep_spec((256, 256)), _rep_spec((1, 256)),
              _rep_spec((256, 128))],
    out_specs=_prow_spec(),
    out_shape=jax.ShapeDtypeStruct((PK, 128), jnp.float32),
)


def _tc_final_body(acc_ref, h2s_ref, dinv_ref, b2_ref, o_ref):
    o_ref[...] = ((acc_ref[0] + acc_ref[1] + h2s_ref[...])
                  * dinv_ref[...] + b2_ref[...])


_tc_final = pl.pallas_call(
    _tc_final_body,
    grid=(TGRID,),
    in_specs=[_pair_spec(), _prow_spec(), _prow_spec(),
              _rep_spec((1, 128))],
    out_specs=_prow_spec(),
    out_shape=jax.ShapeDtypeStruct((PK, 128), jnp.float32),
)


def kernel(x, edge_index, W1, b1, W2, b2):
    src = edge_index[0].astype(jnp.int32)
    dst = edge_index[1].astype(jnp.int32)
    src2d = src.reshape(N_EDGES // CHUNK, CHUNK)
    dst2d = dst.reshape(N_EDGES // CHUNK, CHUNK)

    ones_chunk = jnp.ones((CHUNK,), jnp.float32)
    z1 = jnp.zeros((NPAD,), jnp.float32)
    zn = jnp.zeros((NPAD, FH), jnp.float32)

    NPK = N_NODES * FH // 128
    xpa = jnp.pad(x[:, :FH].reshape(NPK, 128), ((0, PK - NPK), (0, 0)))
    xpb = jnp.pad(x[:, FEAT - FH:].reshape(NPK, 128),
                  ((0, PK - NPK), (0, 0)))

    deg16 = _sc_degree(dst2d, ones_chunk, z1)
    dinv16, xa, xb = _tc_prep(deg16.reshape(NC, PK, 128), xpa, xpb)

    acc1 = _sc_agg_l1(src2d, dst2d, xa.reshape(NPAD, FH),
                      xb.reshape(NPAD, FH), zn)

    w1p = jnp.concatenate(
        [W1[:FH], jnp.zeros((2 * FH - FEAT, HID), W1.dtype), W1[FH:]],
        axis=0)
    w2p = jnp.pad(W2, ((0, 0), (0, FH - OUT)))
    eye8 = jnp.eye(8, dtype=jnp.float32)
    w1big = jnp.einsum("hfgp,kK->hkfgKp", w1p.reshape(2, FH, 2, FH),
                       eye8).reshape(256, 256)
    w2big = jnp.einsum("gpo,kK->gkpKo", w2p.reshape(2, FH, FH),
                       eye8).reshape(256, 128)
    b1big = jnp.broadcast_to(b1.reshape(2, 1, FH), (2, 8, FH)).reshape(1, 256)
    b2big = jnp.broadcast_to(jnp.pad(b2, (0, FH - OUT)).reshape(1, 1, FH),
                             (1, 8, FH)).reshape(1, 128)

    h2s = _tc_dense(acc1.reshape(NC, PK, 128), xa, xb, dinv16,
                    w1big, b1big, w2big)

    acc2 = _sc_agg_l2(src2d, dst2d, h2s.reshape(NPAD, FH), zn)

    outp = _tc_final(acc2.reshape(NC, PK, 128), h2s, dinv16, b2big)
    return (outp[:NPK].reshape(NPK, 8, FH)[:, :, :OUT]
            .reshape(N_NODES, OUT))
